# baseline (speedup 1.0000x reference)
import functools

import jax
import jax.numpy as jnp
from jax import lax
from jax.experimental import pallas as pl
from jax.experimental.pallas import tpu as pltpu
from jax.experimental.pallas import tpu_sc as plsc

_NC = 2
_NS = 16
_NW = _NC * _NS

_B = 16384
_D = 64
_BPW = _B // _NW
_CHUNK = 128
_NCHUNK = _BPW // _CHUNK


def _gather_body(user_hbm, item_hbm, ut_hbm, it_hbm, ue_out, ie_out,
                 uidx_v, urows_v, iidx_v, irows_v, usem, isem):
    wid = lax.axis_index("s") * _NC + lax.axis_index("c")
    base = wid * _BPW
    pltpu.sync_copy(user_hbm.at[wid], uidx_v)
    pltpu.sync_copy(item_hbm.at[wid], iidx_v)
    copies = []
    for j in range(_NCHUNK):
        copies.append(pltpu.async_copy(
            ut_hbm.at[uidx_v.at[j]], urows_v.at[pl.ds(j * _CHUNK, _CHUNK)],
            usem))
        copies.append(pltpu.async_copy(
            it_hbm.at[iidx_v.at[j]], irows_v.at[pl.ds(j * _CHUNK, _CHUNK)],
            isem))
    for c in copies:
        c.wait()
    pltpu.sync_copy(urows_v, ue_out.at[pl.ds(base, _BPW)])
    pltpu.sync_copy(irows_v, ie_out.at[pl.ds(base, _BPW)])


def _sc_gather():
    return pl.kernel(
        _gather_body,
        mesh=plsc.VectorSubcoreMesh(core_axis_name="c", subcore_axis_name="s"),
        compiler_params=pltpu.CompilerParams(use_tc_tiling_on_sc=False),
        out_type=[
            jax.ShapeDtypeStruct((_B, _D), jnp.float32),
            jax.ShapeDtypeStruct((_B, _D), jnp.float32),
        ],
        scratch_types=[
            pltpu.VMEM((_NCHUNK, _CHUNK), jnp.int32),
            pltpu.VMEM((_BPW, _D), jnp.float32),
            pltpu.VMEM((_NCHUNK, _CHUNK), jnp.int32),
            pltpu.VMEM((_BPW, _D), jnp.float32),
            pltpu.SemaphoreType.DMA,
            pltpu.SemaphoreType.DMA,
        ],
    )


def _mlp_body(ue, ie, w1a, w1b, b1, w2, b2, w3, b3, out):
    dot = functools.partial(jnp.dot, preferred_element_type=jnp.float32,
                            precision=lax.Precision.HIGHEST)
    x = dot(ue[...], w1a[...]) + dot(ie[...], w1b[...]) + b1[...]
    x = jnp.maximum(x, 0.0)
    h = jnp.maximum(dot(x, w2[...]) + b2[...], 0.0)
    out[...] = dot(h, w3[...]) + b3[...]


def kernel(user, item, user_table, item_table, W1, b1, W2, b2, W3, b3):
    user = jnp.asarray(user, jnp.int32).reshape(_NW, _NCHUNK, _CHUNK)
    item = jnp.asarray(item, jnp.int32).reshape(_NW, _NCHUNK, _CHUNK)
    ue, ie = _sc_gather()(user, item, user_table, item_table)

    bs = 2048
    grid = (_B // bs,)
    full = lambda r, c: pl.BlockSpec((r, c), lambda i: (0, 0))
    out = pl.pallas_call(
        _mlp_body,
        grid=grid,
        in_specs=[
            pl.BlockSpec((bs, _D), lambda i: (i, 0)),
            pl.BlockSpec((bs, _D), lambda i: (i, 0)),
            full(_D, 64), full(_D, 64), full(1, 64),
            full(64, 32), full(1, 32),
            full(32, 1), full(1, 1),
        ],
        out_specs=pl.BlockSpec((bs, 1), lambda i: (i, 0)),
        out_shape=jax.ShapeDtypeStruct((_B, 1), jnp.float32),
    )(ue, ie, W1[:_D], W1[_D:], b1.reshape(1, 64),
      W2, b2.reshape(1, 32), W3, b3.reshape(1, 1))
    return out


# speedup vs baseline: 1.5217x; 1.5217x over previous
import functools

import jax
import jax.numpy as jnp
from jax import lax
from jax.experimental import pallas as pl
from jax.experimental.pallas import tpu as pltpu
from jax.experimental.pallas import tpu_sc as plsc

_NC = 2
_NS = 16
_NW = _NC * _NS

_B = 16384
_D = 64
_V = 1000000
_VT = (_V + 7) // 8
_BPW = _B // _NW
_CHUNK = 64
_NCHUNK = _BPW // _CHUNK


def _gather_one(idx_ref, tab_ref, out_hbm, base, idx_v, obuf_v, sem):
    pltpu.sync_copy(idx_ref, idx_v)

    def chunk_body(j, _):
        copies = []
        for g in range(_CHUNK // 16):
            v = idx_v[j, pl.ds(g * 16, 16)]
            for k in range(16):
                copies.append(pltpu.async_copy(
                    tab_ref.at[pl.ds(v[k], 1)],
                    obuf_v.at[pl.ds(g * 16 + k, 1)], sem))
        for c in copies:
            c.wait()
        pltpu.sync_copy(obuf_v, out_hbm.at[pl.ds(base + j * _CHUNK, _CHUNK)])
        return ()

    lax.fori_loop(0, _NCHUNK, chunk_body, ())


def _gather_body(user_hbm, item_hbm, ut_hbm, it_hbm, ue_out, ie_out,
                 idx_v, obuf_v, sem):
    wid = lax.axis_index("s") * _NC + lax.axis_index("c")
    base = wid * _BPW
    _gather_one(user_hbm.at[wid], ut_hbm, ue_out, base, idx_v, obuf_v, sem)
    _gather_one(item_hbm.at[wid], it_hbm, ie_out, base, idx_v, obuf_v, sem)


def _sc_gather():
    return pl.kernel(
        _gather_body,
        mesh=plsc.VectorSubcoreMesh(core_axis_name="c", subcore_axis_name="s"),
        out_type=[
            jax.ShapeDtypeStruct((_B, _D), jnp.float32),
            jax.ShapeDtypeStruct((_B, _D), jnp.float32),
        ],
        scratch_types=[
            pltpu.VMEM((_NCHUNK, _CHUNK), jnp.int32),
            pltpu.VMEM((_CHUNK, _D), jnp.float32),
            pltpu.SemaphoreType.DMA,
        ],
    )


def _mlp_body(ue, ie, w1a, w1b, b1, w2, b2, w3, b3, out):
    dot = functools.partial(jnp.dot, preferred_element_type=jnp.float32,
                            precision=lax.Precision.HIGHEST)
    x = dot(ue[...], w1a[...]) + dot(ie[...], w1b[...]) + b1[...]
    x = jnp.maximum(x, 0.0)
    h = jnp.maximum(dot(x, w2[...]) + b2[...], 0.0)
    out[...] = dot(h, w3[...]) + b3[...]


def kernel(user, item, user_table, item_table, W1, b1, W2, b2, W3, b3):
    user = jnp.asarray(user, jnp.int32).reshape(_NW, _NCHUNK, _CHUNK)
    item = jnp.asarray(item, jnp.int32).reshape(_NW, _NCHUNK, _CHUNK)
    ue, ie = _sc_gather()(user, item, user_table, item_table)

    bs = 2048
    grid = (_B // bs,)
    full = lambda r, c: pl.BlockSpec((r, c), lambda i: (0, 0))
    out = pl.pallas_call(
        _mlp_body,
        grid=grid,
        in_specs=[
            pl.BlockSpec((bs, _D), lambda i: (i, 0)),
            pl.BlockSpec((bs, _D), lambda i: (i, 0)),
            full(_D, 64), full(_D, 64), full(1, 64),
            full(64, 32), full(1, 32),
            full(32, 1), full(1, 1),
        ],
        out_specs=pl.BlockSpec((bs, 1), lambda i: (i, 0)),
        out_shape=jax.ShapeDtypeStruct((_B, 1), jnp.float32),
    )(ue, ie, W1[:_D], W1[_D:], b1.reshape(1, 64),
      W2, b2.reshape(1, 32), W3, b3.reshape(1, 1))
    return out


# speedup vs baseline: 1.6052x; 1.0549x over previous
import functools

import jax
import jax.numpy as jnp
from jax import lax
from jax.experimental import pallas as pl
from jax.experimental.pallas import tpu as pltpu
from jax.experimental.pallas import tpu_sc as plsc

_NC = 2
_NS = 16
_NW = _NC * _NS

_B = 16384
_D = 64
_V = 1000000
_VT = (_V + 7) // 8
_BPW = _B // _NW
_CHUNK = 64
_NCHUNK = _BPW // _CHUNK


def _gather_one(idx_hbm, tab_ref, out_hbm, base, idx_v, obuf_v, sem):
    pltpu.sync_copy(idx_hbm.at[pl.ds(base, _BPW)], idx_v)

    def chunk_body(j, _):
        copies = []
        for g in range(_CHUNK // 16):
            v = idx_v[pl.ds(j * _CHUNK + g * 16, 16)]
            for k in range(16):
                copies.append(pltpu.async_copy(
                    tab_ref.at[pl.ds(v[k], 1)],
                    obuf_v.at[pl.ds(g * 16 + k, 1)], sem))
        for c in copies:
            c.wait()
        pltpu.sync_copy(obuf_v, out_hbm.at[pl.ds(base + j * _CHUNK, _CHUNK)])
        return ()

    lax.fori_loop(0, _NCHUNK, chunk_body, ())


def _gather_body(user_hbm, item_hbm, ut_hbm, it_hbm, ue_out, ie_out,
                 idx_v, obuf_v, sem):
    wid = lax.axis_index("s") * _NC + lax.axis_index("c")
    base = wid * _BPW
    _gather_one(user_hbm, ut_hbm, ue_out, base, idx_v, obuf_v, sem)
    _gather_one(item_hbm, it_hbm, ie_out, base, idx_v, obuf_v, sem)


def _sc_gather():
    return pl.kernel(
        _gather_body,
        mesh=plsc.VectorSubcoreMesh(core_axis_name="c", subcore_axis_name="s"),
        out_type=[
            jax.ShapeDtypeStruct((_B, _D), jnp.float32),
            jax.ShapeDtypeStruct((_B, _D), jnp.float32),
        ],
        scratch_types=[
            pltpu.VMEM((_BPW,), jnp.int32),
            pltpu.VMEM((_CHUNK, _D), jnp.float32),
            pltpu.SemaphoreType.DMA,
        ],
    )


def _mlp_body(ue, ie, w1a, w1b, b1, w2, b2, w3, b3, out):
    dot = functools.partial(jnp.dot, preferred_element_type=jnp.float32)
    x = dot(ue[...], w1a[...]) + dot(ie[...], w1b[...]) + b1[...]
    x = jnp.maximum(x, 0.0)
    h = jnp.maximum(dot(x, w2[...]) + b2[...], 0.0)
    out[...] = dot(h, w3[...]) + b3[...]


def kernel(user, item, user_table, item_table, W1, b1, W2, b2, W3, b3):
    user = jnp.asarray(user, jnp.int32)
    item = jnp.asarray(item, jnp.int32)
    ue, ie = _sc_gather()(user, item, user_table, item_table)

    bs = 2048
    grid = (_B // bs,)
    full = lambda r, c: pl.BlockSpec((r, c), lambda i: (0, 0))
    out = pl.pallas_call(
        _mlp_body,
        grid=grid,
        in_specs=[
            pl.BlockSpec((bs, _D), lambda i: (i, 0)),
            pl.BlockSpec((bs, _D), lambda i: (i, 0)),
            full(_D, 64), full(_D, 64), full(1, 64),
            full(64, 32), full(1, 32),
            full(32, 1), full(1, 1),
        ],
        out_specs=pl.BlockSpec((bs, 1), lambda i: (i, 0)),
        out_shape=jax.ShapeDtypeStruct((_B, 1), jnp.float32),
    )(ue, ie, W1[:_D], W1[_D:], b1.reshape(1, 64),
      W2, b2.reshape(1, 32), W3, b3.reshape(1, 1))
    return out


# speedup vs baseline: 1.6102x; 1.0031x over previous
import functools

import jax
import jax.numpy as jnp
from jax import lax
from jax.experimental import pallas as pl
from jax.experimental.pallas import tpu as pltpu
from jax.experimental.pallas import tpu_sc as plsc

_NC = 2
_NS = 16
_NW = _NC * _NS

_B = 16384
_D = 64
_V = 1000000
_VT = (_V + 7) // 8
_BPW = _B // _NW
_CHUNK = 64
_NCHUNK = _BPW // _CHUNK


def _gather_one(idx_hbm, tab_ref, out_hbm, base, idx_v, obuf_v, sem):
    pltpu.sync_copy(idx_hbm.at[pl.ds(base, _BPW)], idx_v)

    def chunk_body(j, _):
        copies = []
        for g in range(_CHUNK // 16):
            v = idx_v[pl.ds(j * _CHUNK + g * 16, 16)]
            for k in range(16):
                copies.append(pltpu.async_copy(
                    tab_ref.at[pl.ds(v[k], 1)],
                    obuf_v.at[pl.ds(g * 16 + k, 1)], sem))
        for c in copies:
            c.wait()
        pltpu.sync_copy(obuf_v, out_hbm.at[pl.ds(base + j * _CHUNK, _CHUNK)])
        return ()

    lax.fori_loop(0, _NCHUNK, chunk_body, ())


def _gather_body(user_hbm, item_hbm, ut_hbm, it_hbm, ue_out, ie_out,
                 idx_v, obuf_v, sem):
    wid = lax.axis_index("s") * _NC + lax.axis_index("c")
    base = wid * _BPW
    _gather_one(user_hbm, ut_hbm, ue_out, base, idx_v, obuf_v, sem)
    _gather_one(item_hbm, it_hbm, ie_out, base, idx_v, obuf_v, sem)


def _sc_gather():
    return pl.kernel(
        _gather_body,
        mesh=plsc.VectorSubcoreMesh(core_axis_name="c", subcore_axis_name="s"),
        compiler_params=pltpu.CompilerParams(use_tc_tiling_on_sc=True),
        out_type=[
            jax.ShapeDtypeStruct((_B, _D), jnp.float32),
            jax.ShapeDtypeStruct((_B, _D), jnp.float32),
        ],
        scratch_types=[
            pltpu.VMEM((_BPW,), jnp.int32),
            pltpu.VMEM((_CHUNK, _D), jnp.float32),
            pltpu.SemaphoreType.DMA,
        ],
    )


def _mlp_body(ue, ie, w1a, w1b, b1, w2, b2, w3, b3, out):
    dot = functools.partial(jnp.dot, preferred_element_type=jnp.float32)
    x = dot(ue[...], w1a[...]) + dot(ie[...], w1b[...]) + b1[...]
    x = jnp.maximum(x, 0.0)
    h = jnp.maximum(dot(x, w2[...]) + b2[...], 0.0)
    out[...] = dot(h, w3[...]) + b3[...]


def kernel(user, item, user_table, item_table, W1, b1, W2, b2, W3, b3):
    user = jnp.asarray(user, jnp.int32)
    item = jnp.asarray(item, jnp.int32)
    ue, ie = _sc_gather()(user, item, user_table, item_table)

    bs = 2048
    grid = (_B // bs,)
    full = lambda r, c: pl.BlockSpec((r, c), lambda i: (0, 0))
    out = pl.pallas_call(
        _mlp_body,
        grid=grid,
        in_specs=[
            pl.BlockSpec((bs, _D), lambda i: (i, 0)),
            pl.BlockSpec((bs, _D), lambda i: (i, 0)),
            full(_D, 64), full(_D, 64), full(1, 64),
            full(64, 32), full(1, 32),
            full(32, 1), full(1, 1),
        ],
        out_specs=pl.BlockSpec((bs, 1), lambda i: (i, 0)),
        out_shape=jax.ShapeDtypeStruct((_B, 1), jnp.float32),
    )(ue, ie, W1[:_D], W1[_D:], b1.reshape(1, 64),
      W2, b2.reshape(1, 32), W3, b3.reshape(1, 1))
    return out
